# baseline (speedup 1.0000x reference)
import functools

import jax
import jax.numpy as jnp
from jax.experimental import pallas as pl
from jax.experimental.pallas import tpu as pltpu


def _round_up(n, m):
    return ((n + m - 1) // m) * m


def _mlp_exp_kernel(x_ref, w0, b0, w1, b1, w2, b2, w3, b3, e_ref, st_ref,
                    *, nbj, tb, num_valid):
    h = x_ref[...].astype(jnp.bfloat16)
    for w_r, b_r in ((w0, b0), (w1, b1), (w2, b2)):
        z = jnp.dot(h, w_r[...].astype(jnp.bfloat16),
                    preferred_element_type=jnp.float32) + b_r[...]
        h = jnp.maximum(z, 0.0).astype(jnp.bfloat16)
    l = jnp.dot(h, w3[...].astype(jnp.bfloat16),
                preferred_element_type=jnp.float32) + b3[...]
    if num_valid is not None:
        t = pl.program_id(0) * nbj + pl.program_id(1)
        rows = jax.lax.broadcasted_iota(jnp.int32, l.shape, 0) + t * tb
        l = jnp.where(rows < num_valid, l, -jnp.inf)
    m = jnp.max(l, axis=0, keepdims=True)
    e = jnp.exp(l - m)
    s = jnp.sum(e, axis=0, keepdims=True)
    e_ref[...] = e
    st_ref[...] = jnp.concatenate([m, s] + [jnp.zeros_like(m)] * 6,
                                  axis=0)[None]


def _rescale_kernel(e_ref, st_ref, o_ref, *, nbj):
    st = st_ref[...]
    m_all = st[:, 0, :]
    s_all = st[:, 1, :]
    g_max = jnp.max(m_all, axis=0, keepdims=True)
    g_sum = jnp.sum(s_all * jnp.exp(m_all - g_max), axis=0, keepdims=True)
    t = pl.program_id(0) * nbj + pl.program_id(1)
    m_t = st_ref[pl.ds(t, 1)][0][0:1, :]
    o_ref[...] = e_ref[...] * (jnp.exp(m_t - g_max) / g_sum)


def kernel(x, p0, p1, p2, p3, p4, p5, p6, p7):
    batch, state_space = x.shape
    a_pad = p6.shape[1]
    action_space = 16

    cores, nbj = 2, 4
    nb = cores * nbj
    batch_p = _round_up(batch, 8 * nb)
    tb = batch_p // nb
    if batch_p != batch:
        x = jnp.pad(x, ((0, batch_p - batch), (0, 0)))
        num_valid = batch
    else:
        num_valid = None

    params = (p0, p1, p2, p3, p4, p5, p6, p7)
    vmem_full = pl.BlockSpec(memory_space=pltpu.MemorySpace.VMEM)

    flops = 0
    for i in range(4):
        k, n = params[2 * i].shape
        flops += 2 * batch_p * k * n
    wb_bytes = sum(int(a.size) * 4 for a in params)
    cost1 = pl.CostEstimate(
        flops=flops,
        transcendentals=batch_p * a_pad,
        bytes_accessed=batch_p * state_space * 4 + wb_bytes
        + batch_p * a_pad * 4 + nb * 8 * a_pad * 4)

    e, st = pl.pallas_call(
        functools.partial(_mlp_exp_kernel, nbj=nbj, tb=tb, num_valid=num_valid),
        grid=(cores, nbj),
        in_specs=[pl.BlockSpec((tb, state_space), lambda i, j: (i * nbj + j, 0))]
        + [vmem_full] * 8,
        out_specs=[
            pl.BlockSpec((tb, a_pad), lambda i, j: (i * nbj + j, 0)),
            pl.BlockSpec((1, 8, a_pad), lambda i, j: (i * nbj + j, 0, 0)),
        ],
        out_shape=[
            jax.ShapeDtypeStruct((batch_p, a_pad), jnp.float32),
            jax.ShapeDtypeStruct((nb, 8, a_pad), jnp.float32),
        ],
        compiler_params=pltpu.CompilerParams(
            dimension_semantics=("parallel", "arbitrary")),
        cost_estimate=cost1,
    )(x, *params)

    probs_p = pl.pallas_call(
        functools.partial(_rescale_kernel, nbj=nbj),
        grid=(cores, nbj),
        in_specs=[
            pl.BlockSpec((tb, a_pad), lambda i, j: (i * nbj + j, 0)),
            vmem_full,
        ],
        out_specs=pl.BlockSpec((tb, a_pad), lambda i, j: (i * nbj + j, 0)),
        out_shape=jax.ShapeDtypeStruct((batch_p, a_pad), jnp.float32),
        compiler_params=pltpu.CompilerParams(
            dimension_semantics=("parallel", "arbitrary")),
        cost_estimate=pl.CostEstimate(
            flops=batch_p * a_pad, transcendentals=nb * a_pad * 2,
            bytes_accessed=2 * batch_p * a_pad * 4),
    )(e, st)

    return probs_p[:batch, :action_space]


# speedup vs baseline: 1.2030x; 1.2030x over previous
import functools

import jax
import jax.numpy as jnp
from jax.experimental import pallas as pl
from jax.experimental.pallas import tpu as pltpu


def _round_up(n, m):
    return ((n + m - 1) // m) * m


def _actor_kernel(x_ref, w0, b0, w1, b1, w2, b2, w3, b3, o_ref,
                  e_scr, m_scr, s_scr, *, nb, tb, num_valid, a_out):
    j = pl.program_id(0)
    h = x_ref[...].astype(jnp.bfloat16)
    for w_r, b_r in ((w0, b0), (w1, b1), (w2, b2)):
        z = jnp.dot(h, w_r[...].astype(jnp.bfloat16),
                    preferred_element_type=jnp.float32) + b_r[...]
        h = jnp.maximum(z, 0.0).astype(jnp.bfloat16)
    l = jnp.dot(h, w3[...].astype(jnp.bfloat16),
                preferred_element_type=jnp.float32) + b3[...]
    if num_valid is not None:
        rows = jax.lax.broadcasted_iota(jnp.int32, l.shape, 0) + j * tb
        l = jnp.where(rows < num_valid, l, -jnp.inf)
    m = jnp.max(l, axis=0, keepdims=True)
    e = jnp.exp(l - m)
    s = jnp.sum(e, axis=0, keepdims=True)
    e_scr[pl.ds(j * tb, tb), :] = e
    m_scr[pl.ds(j, 1), :] = m
    s_scr[pl.ds(j, 1), :] = s

    @pl.when(j == nb - 1)
    def _finalize():
        m_all = m_scr[...]
        s_all = s_scr[...]
        g_max = jnp.max(m_all, axis=0, keepdims=True)
        g_sum = jnp.sum(s_all * jnp.exp(m_all - g_max), axis=0, keepdims=True)
        fac = jnp.exp(m_all - g_max) / g_sum
        for t in range(nb):
            o_ref[pl.ds(t * tb, tb), :] = (
                e_scr[pl.ds(t * tb, tb), :] * fac[t:t + 1, :])[:, :a_out]


def kernel(x, p0, p1, p2, p3, p4, p5, p6, p7):
    batch, state_space = x.shape
    a_pad = p6.shape[1]
    a_out = 16

    nb = 8
    batch_p = _round_up(batch, 8 * nb)
    tb = batch_p // nb
    if batch_p != batch:
        x = jnp.pad(x, ((0, batch_p - batch), (0, 0)))
        num_valid = batch
    else:
        num_valid = None

    params = (p0, p1, p2, p3, p4, p5, p6, p7)
    vmem_full = pl.BlockSpec(memory_space=pltpu.MemorySpace.VMEM)

    flops = 0
    for i in range(4):
        k, n = params[2 * i].shape
        flops += 2 * batch_p * k * n
    wb_bytes = sum(int(a.size) * 4 for a in params)
    cost = pl.CostEstimate(
        flops=flops,
        transcendentals=batch_p * a_pad,
        bytes_accessed=batch_p * state_space * 4 + wb_bytes
        + batch_p * a_out * 4)

    probs_p = pl.pallas_call(
        functools.partial(_actor_kernel, nb=nb, tb=tb, num_valid=num_valid,
                          a_out=a_out),
        grid=(nb,),
        in_specs=[pl.BlockSpec((tb, state_space), lambda j: (j, 0))]
        + [vmem_full] * 8,
        out_specs=pl.BlockSpec((batch_p, a_out), lambda j: (0, 0)),
        out_shape=jax.ShapeDtypeStruct((batch_p, a_out), jnp.float32),
        scratch_shapes=[
            pltpu.VMEM((batch_p, a_pad), jnp.float32),
            pltpu.VMEM((nb, a_pad), jnp.float32),
            pltpu.VMEM((nb, a_pad), jnp.float32),
        ],
        compiler_params=pltpu.CompilerParams(
            dimension_semantics=("arbitrary",)),
        cost_estimate=cost,
    )(x, *params)

    return probs_p[:batch]
